# baseline (speedup 1.0000x reference)
import functools

import jax
import jax.numpy as jnp
from jax import lax
from jax.experimental import pallas as pl
from jax.experimental.pallas import tpu as pltpu
from jax.experimental.pallas import tpu_sc as plsc

_NC = 2
_NS = 16
_NW = _NC * _NS
_C = 32


def kernel(inputs, pos_table):
    B, S, D = inputs.shape
    rows_w = S // _NW
    n_chunks = rows_w // _C
    chunk = _C * D
    x_flat = inputs.reshape(B * S * D)
    p_flat = pos_table.reshape(S * D)

    mesh = plsc.VectorSubcoreMesh(core_axis_name="c", subcore_axis_name="s")

    @functools.partial(
        pl.kernel,
        out_type=jax.ShapeDtypeStruct((B * S * D,), jnp.float32),
        mesh=mesh,
        scratch_types=[
            pltpu.VMEM((chunk,), jnp.float32),
            pltpu.VMEM((chunk,), jnp.float32),
        ],
    )
    def sc_add(x_hbm, p_hbm, o_hbm, xv, pv):
        wid = lax.axis_index("s") * _NC + lax.axis_index("c")
        base = wid * rows_w * D

        def chunk_body(c, _):
            p_off = base + c * chunk
            pltpu.sync_copy(p_hbm.at[pl.ds(p_off, chunk)], pv)
            for b in range(B):
                x_off = b * S * D + p_off
                pltpu.sync_copy(x_hbm.at[pl.ds(x_off, chunk)], xv)

                def vec_body(i, _):
                    sl = pl.ds(i * 16, 16)
                    xv[sl] = xv[sl] + pv[sl]
                    return 0

                lax.fori_loop(0, chunk // 16, vec_body, 0)
                pltpu.sync_copy(xv, o_hbm.at[pl.ds(x_off, chunk)])
            return 0

        lax.fori_loop(0, n_chunks, chunk_body, 0)

    return sc_add(x_flat, p_flat).reshape(B, S, D)


# speedup vs baseline: 1.5351x; 1.5351x over previous
import functools

import jax
import jax.numpy as jnp
from jax import lax
from jax.experimental import pallas as pl
from jax.experimental.pallas import tpu as pltpu
from jax.experimental.pallas import tpu_sc as plsc

_NC = 2
_NS = 16
_NW = _NC * _NS
_C = 32
_U = 8


def kernel(inputs, pos_table):
    B, S, D = inputs.shape
    rows_w = S // _NW
    n_chunks = rows_w // _C
    chunk = _C * D
    n_g = n_chunks * B
    x_flat = inputs.reshape(B * S * D)
    p_flat = pos_table.reshape(S * D)

    mesh = plsc.VectorSubcoreMesh(core_axis_name="c", subcore_axis_name="s")

    @functools.partial(
        pl.kernel,
        out_type=jax.ShapeDtypeStruct((B * S * D,), jnp.float32),
        mesh=mesh,
        scratch_types=[
            pltpu.VMEM((2, chunk), jnp.float32),
            pltpu.VMEM((2, chunk), jnp.float32),
            pltpu.SemaphoreType.DMA,
            pltpu.SemaphoreType.DMA,
            pltpu.SemaphoreType.DMA,
            pltpu.SemaphoreType.DMA,
            pltpu.SemaphoreType.DMA,
            pltpu.SemaphoreType.DMA,
        ],
    )
    def sc_add(x_hbm, p_hbm, o_hbm, xv, pv, sx0, sx1, sp0, sp1, so0, so1):
        wid = lax.axis_index("s") * _NC + lax.axis_index("c")
        base = wid * rows_w * D
        sx = (sx0, sx1)
        sp = (sp0, sp1)
        so = (so0, so1)

        def x_off(g):
            c, b = divmod(g, B)
            return b * S * D + base + c * chunk

        def start_x(g):
            return pltpu.async_copy(
                x_hbm.at[pl.ds(x_off(g), chunk)], xv.at[g % 2], sx[g % 2]
            )

        def start_p(c):
            return pltpu.async_copy(
                p_hbm.at[pl.ds(base + c * chunk, chunk)], pv.at[c % 2], sp[c % 2]
            )

        h_p = start_p(0)
        h_x = start_x(0)
        h_out = [None, None]
        hp_next = None

        for g in range(n_g):
            c, b = divmod(g, B)
            if b == 0 and c + 1 < n_chunks:
                hp_next = start_p(c + 1)
            h_x_next = None
            if g + 1 < n_g:
                if h_out[(g + 1) % 2] is not None:
                    h_out[(g + 1) % 2].wait()
                    h_out[(g + 1) % 2] = None
                h_x_next = start_x(g + 1)
            if b == 0:
                h_p.wait()
                h_p = hp_next
            h_x.wait()
            h_x = h_x_next

            xb = xv.at[g % 2]
            pb = pv.at[c % 2]

            def vec_body(i, _, xb=xb, pb=pb):
                off = i * (16 * _U)
                for u in range(_U):
                    sl = pl.ds(off + u * 16, 16)
                    xb[sl] = xb[sl] + pb[sl]
                return 0

            lax.fori_loop(0, chunk // (16 * _U), vec_body, 0)

            h_out[g % 2] = pltpu.async_copy(
                xv.at[g % 2], o_hbm.at[pl.ds(x_off(g), chunk)], so[g % 2]
            )

        for h in h_out:
            if h is not None:
                h.wait()

    return sc_add(x_flat, p_flat).reshape(B, S, D)


# speedup vs baseline: 4.6993x; 3.0612x over previous
import functools

import jax
import jax.numpy as jnp
from jax import lax
from jax.experimental import pallas as pl
from jax.experimental.pallas import tpu as pltpu
from jax.experimental.pallas import tpu_sc as plsc

_NC = 2
_NS = 16
_NW = _NC * _NS
_C = 16
_NB = 4


def kernel(inputs, pos_table):
    B, S, D = inputs.shape
    rows_w = S // _NW
    n_chunks = rows_w // _C
    n_units = n_chunks * B
    units_per_iter = 2 * B
    n_iters = n_units // units_per_iter

    mesh = plsc.VectorSubcoreMesh(core_axis_name="c", subcore_axis_name="s")

    @functools.partial(
        pl.kernel,
        out_type=jax.ShapeDtypeStruct((B, S, D), jnp.float32),
        mesh=mesh,
        scratch_types=[
            pltpu.VMEM((_NB, _C, D), jnp.float32),
            pltpu.VMEM((2, _C, D), jnp.float32),
        ]
        + [pltpu.SemaphoreType.DMA] * (_NB + _NB + 2),
    )
    def sc_add(x_hbm, p_hbm, o_hbm, xv, pv, *sems):
        sx = sems[:_NB]
        so = sems[_NB:2 * _NB]
        sp = sems[2 * _NB:]
        wid = lax.axis_index("s") * _NC + lax.axis_index("c")
        row0 = wid * rows_w

        def start_x(cidx, b, slot):
            return pltpu.async_copy(
                x_hbm.at[b, pl.ds(row0 + cidx * _C, _C), :], xv.at[slot], sx[slot]
            )

        def start_p(cidx, pslot):
            return pltpu.async_copy(
                p_hbm.at[pl.ds(row0 + cidx * _C, _C), :], pv.at[pslot], sp[pslot]
            )

        def start_out(cidx, b, slot):
            return pltpu.async_copy(
                xv.at[slot], o_hbm.at[b, pl.ds(row0 + cidx * _C, _C), :], so[slot]
            )

        def wait_chunk(sem):
            pltpu.make_async_copy(
                x_hbm.at[0, pl.ds(row0, _C), :], xv.at[0], sem
            ).wait()

        def compute(xslot, pslot):
            def row_body(r, _):
                xr = xv.at[xslot, r]
                pr = pv.at[pslot, r]
                for j in range(D // 16):
                    sl = pl.ds(j * 16, 16)
                    xr[sl] = xr[sl] + pr[sl]
                return 0

            lax.fori_loop(0, _C, row_body, 0)

        start_p(0, 0)
        for g in range(_NB - 1):
            start_x(g // B, g % B, g % _NB)

        def outer(cc, _):
            for u in range(units_per_iter):
                g = cc * units_per_iter + u
                half = u // B
                b = u % B
                cidx = 2 * cc + half
                xslot = u % _NB
                pslot = half
                if u == 0:
                    wait_chunk(sp[0])
                    start_p(2 * cc + 1, 1)
                if u == B:
                    wait_chunk(sp[1])

                    @pl.when(cc < n_iters - 1)
                    def _():
                        start_p(2 * cc + 2, 0)

                wait_chunk(sx[xslot])
                compute(xslot, pslot)
                start_out(cidx, b, xslot)
                if u == 0:
                    @pl.when(cc > 0)
                    def _():
                        wait_chunk(so[(u + 3) % _NB])
                else:
                    wait_chunk(so[(u + 3) % _NB])
                t = u + _NB - 1
                if t < units_per_iter:
                    start_x(2 * cc + t // B, t % B, t % _NB)
                else:
                    tn = t - units_per_iter

                    @pl.when(cc < n_iters - 1)
                    def _():
                        start_x(2 * (cc + 1) + tn // B, tn % B, t % _NB)
            return 0

        lax.fori_loop(0, n_iters, outer, 0)
        wait_chunk(so[(n_units - 1) % _NB])

    return sc_add(inputs, pos_table)


# speedup vs baseline: 5.4566x; 1.1612x over previous
import functools

import jax
import jax.numpy as jnp
from jax import lax
from jax.experimental import pallas as pl
from jax.experimental.pallas import tpu as pltpu
from jax.experimental.pallas import tpu_sc as plsc

_NC = 2
_NS = 16
_NW = _NC * _NS
_C = 16
_NB = 4


def kernel(inputs, pos_table):
    B, S, D = inputs.shape
    rows_w = S // _NW
    n_chunks = rows_w // _C
    n_units = n_chunks * B
    units_per_iter = 2 * B
    n_iters = n_units // units_per_iter

    mesh = plsc.VectorSubcoreMesh(core_axis_name="c", subcore_axis_name="s")

    @functools.partial(
        pl.kernel,
        out_type=jax.ShapeDtypeStruct((B, S, D), jnp.float32),
        mesh=mesh,
        scratch_types=[
            pltpu.VMEM((_NB, _C, D), jnp.float32),
            pltpu.VMEM((2, _C, D), jnp.float32),
        ]
        + [pltpu.SemaphoreType.DMA] * (_NB + _NB + 2),
    )
    def sc_add(x_hbm, p_hbm, o_hbm, xv, pv, *sems):
        sx = sems[:_NB]
        so = sems[_NB:2 * _NB]
        sp = sems[2 * _NB:]
        wid = lax.axis_index("s") * _NC + lax.axis_index("c")
        row0 = wid * rows_w

        def start_x(cidx, b, slot):
            return pltpu.async_copy(
                x_hbm.at[b, pl.ds(row0 + cidx * _C, _C), :], xv.at[slot], sx[slot]
            )

        def start_p(cidx, pslot):
            return pltpu.async_copy(
                p_hbm.at[pl.ds(row0 + cidx * _C, _C), :], pv.at[pslot], sp[pslot]
            )

        def start_out(cidx, b, slot):
            return pltpu.async_copy(
                xv.at[slot], o_hbm.at[b, pl.ds(row0 + cidx * _C, _C), :], so[slot]
            )

        def wait_chunk(sem):
            pltpu.make_async_copy(
                x_hbm.at[0, pl.ds(row0, _C), :], xv.at[0], sem
            ).wait()

        def compute(xslot, pslot):
            @plsc.parallel_loop(0, _C)
            def row_body(r):
                xr = xv.at[xslot, r]
                pr = pv.at[pslot, r]
                for j in range(D // 16):
                    sl = pl.ds(j * 16, 16)
                    xr[sl] = xr[sl] + pr[sl]

        start_p(0, 0)
        for g in range(_NB - 1):
            start_x(g // B, g % B, g % _NB)

        def outer(cc, _):
            for u in range(units_per_iter):
                g = cc * units_per_iter + u
                half = u // B
                b = u % B
                cidx = 2 * cc + half
                xslot = u % _NB
                pslot = half
                if u == 0:
                    wait_chunk(sp[0])
                    start_p(2 * cc + 1, 1)
                if u == B:
                    wait_chunk(sp[1])

                    @pl.when(cc < n_iters - 1)
                    def _():
                        start_p(2 * cc + 2, 0)

                wait_chunk(sx[xslot])
                compute(xslot, pslot)
                start_out(cidx, b, xslot)
                if u == 0:
                    @pl.when(cc > 0)
                    def _():
                        wait_chunk(so[(u + 3) % _NB])
                else:
                    wait_chunk(so[(u + 3) % _NB])
                t = u + _NB - 1
                if t < units_per_iter:
                    start_x(2 * cc + t // B, t % B, t % _NB)
                else:
                    tn = t - units_per_iter

                    @pl.when(cc < n_iters - 1)
                    def _():
                        start_x(2 * (cc + 1) + tn // B, tn % B, t % _NB)
            return 0

        lax.fori_loop(0, n_iters, outer, 0)
        wait_chunk(so[(n_units - 1) % _NB])

    return sc_add(inputs, pos_table)
